# baseline (device time: 35142 ns/iter reference)
import jax
import jax.numpy as jnp
from jax import lax
from jax.experimental import pallas as pl
from jax.experimental.pallas import tpu as pltpu

M = 2048
P = 1072
F = M - P
R = P - F
F_SIZES = [64] * 14 + [32, 24, 16, 8]
F_OFFS = [sum(F_SIZES[:i]) for i in range(len(F_SIZES))]
assert sum(F_SIZES) == F
KF = len(F_SIZES)
KY = KF + 1


def kernel(x):
    m_per, n = x.shape
    assert m_per == M

    def body(x_ref, out_ref, fbuf, ysend, yrecv, xsend, xrecv, fcopy, own_sem):
        my_x = lax.axis_index("x")
        my_y = lax.axis_index("y")
        y_nbr = (my_x, 1 - my_y)
        x_nbr = (1 - my_x, my_y)

        f_base = my_x * P
        other = (1 - my_y) * M

        barrier_sem = pltpu.get_barrier_semaphore()
        for nbr in (y_nbr, x_nbr):
            pl.semaphore_signal(
                barrier_sem, inc=1,
                device_id=nbr, device_id_type=pl.DeviceIdType.MESH,
            )
        pl.semaphore_wait(barrier_sem, 2)

        y_rdmas = []
        for c, (off, sz) in enumerate(zip(F_OFFS, F_SIZES)):
            rdma = pltpu.make_async_remote_copy(
                src_ref=x_ref.at[pl.ds(f_base + off, sz), :],
                dst_ref=fbuf.at[pl.ds(off, sz), :],
                send_sem=ysend.at[c],
                recv_sem=yrecv.at[c],
                device_id=y_nbr,
                device_id_type=pl.DeviceIdType.MESH,
            )
            rdma.start()
            y_rdmas.append(rdma)
        r_rdma = pltpu.make_async_remote_copy(
            src_ref=x_ref.at[pl.ds(F, R), :],
            dst_ref=out_ref.at[pl.ds(my_y * M + F, R), :],
            send_sem=ysend.at[KF],
            recv_sem=yrecv.at[KF],
            device_id=y_nbr,
            device_id_type=pl.DeviceIdType.MESH,
        )
        r_rdma.start()
        y_rdmas.append(r_rdma)

        own = pltpu.make_async_copy(
            x_ref, out_ref.at[pl.ds(my_y * M, M), :], own_sem
        )
        own.start()

        x_rdmas = []
        copies = []
        for c, (off, sz) in enumerate(zip(F_OFFS, F_SIZES)):
            y_rdmas[c].wait_recv()
            rdma = pltpu.make_async_remote_copy(
                src_ref=fbuf.at[pl.ds(off, sz), :],
                dst_ref=out_ref.at[pl.ds(other + f_base + off, sz), :],
                send_sem=xsend.at[c],
                recv_sem=xrecv.at[c],
                device_id=x_nbr,
                device_id_type=pl.DeviceIdType.MESH,
            )
            rdma.start()
            x_rdmas.append(rdma)
            cp = pltpu.make_async_copy(
                fbuf.at[pl.ds(off, sz), :],
                out_ref.at[pl.ds(other + f_base + off, sz), :],
                fcopy.at[c],
            )
            cp.start()
            copies.append(cp)

        y_rdmas[KF].wait_recv()
        for c in range(KF):
            x_rdmas[c].wait_recv()
        own.wait()
        for c in range(KF):
            copies[c].wait()
        for c in range(KY):
            y_rdmas[c].wait_send()
        for c in range(KF):
            x_rdmas[c].wait_send()

    return pl.pallas_call(
        body,
        out_shape=jax.ShapeDtypeStruct((2 * m_per, n), x.dtype),
        in_specs=[pl.BlockSpec(memory_space=pltpu.MemorySpace.HBM)],
        out_specs=pl.BlockSpec(memory_space=pltpu.MemorySpace.HBM),
        scratch_shapes=[
            pltpu.VMEM((F, n), x.dtype),
            pltpu.SemaphoreType.DMA((KY,)),
            pltpu.SemaphoreType.DMA((KY,)),
            pltpu.SemaphoreType.DMA((KF,)),
            pltpu.SemaphoreType.DMA((KF,)),
            pltpu.SemaphoreType.DMA((KF,)),
            pltpu.SemaphoreType.DMA,
        ],
        compiler_params=pltpu.CompilerParams(collective_id=0),
    )(x)
